# Initial kernel scaffold; baseline (speedup 1.0000x reference)
import functools

import jax
import jax.numpy as jnp
from jax import lax
from jax.experimental import pallas as pl
from jax.experimental.pallas import tpu as pltpu
from jax.experimental.pallas import tpu_sc as plsc

VOCAB = 1000000
EMBED = 32
B = 16384
L = 50

N_TOTAL = B * L
NW = 32
N_PER_W = N_TOTAL // NW
SLICE = 128
N_SLICES = N_PER_W // SLICE
CHUNK_SLICES = 10
CHUNK = CHUNK_SLICES * SLICE
N_CHUNKS = N_SLICES // CHUNK_SLICES

_mesh = plsc.VectorSubcoreMesh(core_axis_name="c", subcore_axis_name="s")


@functools.partial(
    pl.kernel,
    out_type=jax.ShapeDtypeStruct((N_TOTAL, EMBED), jnp.float32),
    mesh=_mesh,
    scratch_types=[
        pltpu.VMEM((N_SLICES, SLICE), jnp.int32),
        pltpu.VMEM((CHUNK, EMBED), jnp.float32),
        pltpu.SemaphoreType.DMA,
    ],
)
def _embed_gather(word_hbm, table_hbm, out_hbm, idx_v, rows_v, gsem):
    wid = lax.axis_index("s") * 2 + lax.axis_index("c")
    pltpu.sync_copy(word_hbm.at[pl.ds(wid * N_SLICES, N_SLICES)], idx_v)
    base = wid * N_PER_W

    @pl.loop(0, N_CHUNKS)
    def _chunk(c):
        handles = []
        for j in range(CHUNK_SLICES):
            h = pltpu.async_copy(
                table_hbm.at[idx_v.at[c * CHUNK_SLICES + j]],
                rows_v.at[pl.ds(j * SLICE, SLICE)],
                gsem,
            )
            handles.append(h)
        for h in handles:
            h.wait()
        pltpu.sync_copy(rows_v, out_hbm.at[pl.ds(base + c * CHUNK, CHUNK)])


def kernel(word, table):
    flat = word.reshape(NW * N_SLICES, SLICE)
    out = _embed_gather(flat, table)
    return out.reshape(B, L, EMBED)


# speedup vs baseline: 1.1055x; 1.1055x over previous
import functools

import jax
import jax.numpy as jnp
from jax import lax
from jax.experimental import pallas as pl
from jax.experimental.pallas import tpu as pltpu
from jax.experimental.pallas import tpu_sc as plsc

VOCAB = 1000000
EMBED = 32
B = 16384
L = 50

N_TOTAL = B * L
NW = 32
N_PER_W = N_TOTAL // NW
SLICE = 128
N_SLICES = N_PER_W // SLICE
CHUNK_SLICES = 10
CHUNK = CHUNK_SLICES * SLICE
N_CHUNKS = N_SLICES // CHUNK_SLICES

_mesh = plsc.VectorSubcoreMesh(core_axis_name="c", subcore_axis_name="s")


@functools.partial(
    pl.kernel,
    out_type=jax.ShapeDtypeStruct((N_TOTAL, EMBED), jnp.float32),
    mesh=_mesh,
    scratch_types=[
        pltpu.VMEM((N_SLICES, SLICE), jnp.int32),
        pltpu.VMEM((CHUNK, EMBED), jnp.float32),
        pltpu.SemaphoreType.DMA,
    ],
    compiler_params=pltpu.CompilerParams(use_tc_tiling_on_sc=False),
)
def _embed_gather(word_hbm, table_hbm, out_hbm, idx_v, rows_v, gsem):
    wid = lax.axis_index("s") * 2 + lax.axis_index("c")
    pltpu.sync_copy(word_hbm.at[pl.ds(wid * N_SLICES, N_SLICES)], idx_v)
    base = wid * N_PER_W

    @pl.loop(0, N_CHUNKS)
    def _chunk(c):
        handles = []
        for j in range(CHUNK_SLICES):
            h = pltpu.async_copy(
                table_hbm.at[idx_v.at[c * CHUNK_SLICES + j]],
                rows_v.at[pl.ds(j * SLICE, SLICE)],
                gsem,
            )
            handles.append(h)
        for h in handles:
            h.wait()
        pltpu.sync_copy(rows_v, out_hbm.at[pl.ds(base + c * CHUNK, CHUNK)])


def kernel(word, table):
    flat = word.reshape(NW * N_SLICES, SLICE)
    out = _embed_gather(flat, table)
    return out.reshape(B, L, EMBED)


# speedup vs baseline: 1.1105x; 1.0045x over previous
import functools

import jax
import jax.numpy as jnp
from jax import lax
from jax.experimental import pallas as pl
from jax.experimental.pallas import tpu as pltpu
from jax.experimental.pallas import tpu_sc as plsc

VOCAB = 1000000
EMBED = 32
B = 16384
L = 50

N_TOTAL = B * L
NW = 32
N_PER_W = N_TOTAL // NW
SLICE = 128
N_SLICES = N_PER_W // SLICE
CHUNK_SLICES = 10
CHUNK = CHUNK_SLICES * SLICE
N_CHUNKS = N_SLICES // CHUNK_SLICES

_mesh = plsc.VectorSubcoreMesh(core_axis_name="c", subcore_axis_name="s")


@functools.partial(
    pl.kernel,
    out_type=jax.ShapeDtypeStruct((N_TOTAL, EMBED), jnp.float32),
    mesh=_mesh,
    scratch_types=[
        pltpu.VMEM((N_SLICES, SLICE), jnp.int32),
        pltpu.VMEM((CHUNK, EMBED), jnp.float32),
        pltpu.VMEM((CHUNK, EMBED), jnp.float32),
        pltpu.SemaphoreType.DMA,
        pltpu.SemaphoreType.DMA,
        pltpu.SemaphoreType.DMA,
        pltpu.SemaphoreType.DMA,
    ],
    compiler_params=pltpu.CompilerParams(use_tc_tiling_on_sc=False),
)
def _embed_gather(word_hbm, table_hbm, out_hbm, idx_v, rows0, rows1,
                  gsem0, gsem1, osem0, osem1):
    wid = lax.axis_index("s") * 2 + lax.axis_index("c")
    pltpu.sync_copy(word_hbm.at[pl.ds(wid * N_SLICES, N_SLICES)], idx_v)
    base = wid * N_PER_W

    def fire_gathers(chunk, buf, sem):
        handles = []
        for j in range(CHUNK_SLICES):
            handles.append(pltpu.async_copy(
                table_hbm.at[idx_v.at[chunk * CHUNK_SLICES + j]],
                buf.at[pl.ds(j * SLICE, SLICE)],
                sem,
            ))
        return handles

    @pl.loop(0, N_CHUNKS, step=2)
    def _chunk(c):
        g0 = fire_gathers(c, rows0, gsem0)
        g1 = fire_gathers(c + 1, rows1, gsem1)
        for h in g0:
            h.wait()
        o0 = pltpu.async_copy(rows0, out_hbm.at[pl.ds(base + c * CHUNK, CHUNK)], osem0)
        for h in g1:
            h.wait()
        o1 = pltpu.async_copy(rows1, out_hbm.at[pl.ds(base + (c + 1) * CHUNK, CHUNK)], osem1)
        o0.wait()
        o1.wait()


def kernel(word, table):
    flat = word.reshape(NW * N_SLICES, SLICE)
    out = _embed_gather(flat, table)
    return out.reshape(B, L, EMBED)


# speedup vs baseline: 1.3986x; 1.2594x over previous
import functools

import jax
import jax.numpy as jnp
from jax import lax
from jax.experimental import pallas as pl
from jax.experimental.pallas import tpu as pltpu
from jax.experimental.pallas import tpu_sc as plsc

VOCAB = 1000000
EMBED = 32
B = 16384
L = 50

N_TOTAL = B * L
NW = 32
B_PER_W = B // NW
N_PER_W = B_PER_W * L
SLICE = 128
P_PER_W = B_PER_W // SLICE
N_CHUNKS = L * P_PER_W

_mesh = plsc.VectorSubcoreMesh(core_axis_name="c", subcore_axis_name="s")


@functools.partial(
    pl.kernel,
    out_type=jax.ShapeDtypeStruct((L, 4, 128, 8, 128), jnp.float32),
    mesh=_mesh,
    scratch_types=[
        pltpu.VMEM((N_PER_W,), jnp.int32),
        pltpu.VMEM((N_CHUNKS, SLICE), jnp.int32),
        pltpu.VMEM((SLICE, EMBED), jnp.float32),
        pltpu.VMEM((SLICE, EMBED), jnp.float32),
        pltpu.VMEM((EMBED, SLICE), jnp.float32),
        pltpu.VMEM((EMBED, SLICE), jnp.float32),
        pltpu.SemaphoreType.DMA,
        pltpu.SemaphoreType.DMA,
        pltpu.SemaphoreType.DMA,
        pltpu.SemaphoreType.DMA,
    ],
    compiler_params=pltpu.CompilerParams(
        use_tc_tiling_on_sc=False, needs_layout_passes=False),
)
def _embed_gather(word_hbm, table_hbm, out_hbm, idx_v, idx2_v,
                  rows0, rows1, tbuf0, tbuf1, gsem0, gsem1, osem0, osem1):
    wid = lax.axis_index("s") * 2 + lax.axis_index("c")
    iota16 = lax.iota(jnp.int32, 16)
    pltpu.sync_copy(word_hbm.at[pl.ds(wid * N_PER_W, N_PER_W)], idx_v)

    @pl.loop(0, L)
    def _regroup(l):
        for p in range(P_PER_W):
            for j in range(8):
                ivec = (iota16 + (SLICE * p + 16 * j)) * L + l
                vals = plsc.load_gather(idx_v, [ivec])
                idx2_v[l * P_PER_W + p, pl.ds(16 * j, 16)] = vals

    p_base = wid * P_PER_W

    def fire_gather(k, buf, sem):
        return pltpu.async_copy(table_hbm.at[idx2_v.at[k]], buf, sem)

    def transpose_and_out(k, buf, tbuf, osem):
        for e in range(EMBED):
            col = jnp.full((16,), e, dtype=jnp.int32)
            for j in range(8):
                vals = plsc.load_gather(buf, [iota16 + 16 * j, col])
                tbuf[e, pl.ds(16 * j, 16)] = vals
        l = k // P_PER_W
        p = p_base + lax.rem(k, P_PER_W)
        handles = []
        for g in range(4):
            handles.append(pltpu.async_copy(
                tbuf.at[pl.ds(8 * g, 8)], out_hbm.at[l, g, p], osem))
        return handles

    @pl.loop(0, N_CHUNKS, step=2)
    def _chunk(k):
        g0 = fire_gather(k, rows0, gsem0)
        g1 = fire_gather(k + 1, rows1, gsem1)
        g0.wait()
        o0 = transpose_and_out(k, rows0, tbuf0, osem0)
        g1.wait()
        o1 = transpose_and_out(k + 1, rows1, tbuf1, osem1)
        for h in o0:
            h.wait()
        for h in o1:
            h.wait()


def kernel(word, table):
    flat_word = word.reshape(N_TOTAL)
    out5 = _embed_gather(flat_word, table)
    return out5.transpose(2, 4, 0, 1, 3).reshape(B, L, EMBED)


# speedup vs baseline: 1.6548x; 1.1832x over previous
import functools

import jax
import jax.numpy as jnp
from jax import lax
from jax.experimental import pallas as pl
from jax.experimental.pallas import tpu as pltpu
from jax.experimental.pallas import tpu_sc as plsc

VOCAB = 1000000
EMBED = 32
B = 16384
L = 50

N_TOTAL = B * L
NW = 32
B_PER_W = B // NW
N_PER_W = B_PER_W * L
SLICE = 128
P_PER_W = B_PER_W // SLICE
N_CHUNKS = L * P_PER_W

_mesh = plsc.VectorSubcoreMesh(core_axis_name="c", subcore_axis_name="s")


@functools.partial(
    pl.kernel,
    out_type=jax.ShapeDtypeStruct((L, 4, 128, 1024), jnp.float32),
    mesh=_mesh,
    scratch_types=[
        pltpu.VMEM((N_PER_W,), jnp.int32),
        pltpu.VMEM((N_CHUNKS, SLICE), jnp.int32),
        pltpu.VMEM((SLICE, EMBED), jnp.float32),
        pltpu.VMEM((SLICE, EMBED), jnp.float32),
        pltpu.VMEM((EMBED * SLICE,), jnp.float32),
        pltpu.VMEM((EMBED * SLICE,), jnp.float32),
        pltpu.SemaphoreType.DMA,
        pltpu.SemaphoreType.DMA,
        pltpu.SemaphoreType.DMA,
        pltpu.SemaphoreType.DMA,
    ],
    compiler_params=pltpu.CompilerParams(
        use_tc_tiling_on_sc=False, needs_layout_passes=False),
)
def _embed_gather(word_hbm, table_hbm, out_hbm, idx_v, idx2_v,
                  rows0, rows1, tbuf0, tbuf1, gsem0, gsem1, osem0, osem1):
    wid = lax.axis_index("s") * 2 + lax.axis_index("c")
    iota16 = lax.iota(jnp.int32, 16)
    pltpu.sync_copy(word_hbm.at[pl.ds(wid * N_PER_W, N_PER_W)], idx_v)

    @pl.loop(0, L)
    def _regroup(l):
        for p in range(P_PER_W):
            for j in range(8):
                ivec = (iota16 + (SLICE * p + 16 * j)) * L + l
                vals = plsc.load_gather(idx_v, [ivec])
                idx2_v[l * P_PER_W + p, pl.ds(16 * j, 16)] = vals

    p_base = wid * P_PER_W

    def fire_gather(k, buf, sem):
        return pltpu.async_copy(table_hbm.at[idx2_v.at[k]], buf, sem)

    siota = iota16 * SLICE

    def transpose_and_out(k, buf, tbuf, osem):
        for c in range(SLICE):
            for h in range(2):
                vals = buf[c, pl.ds(16 * h, 16)]
                ivec = siota + (2048 * h + c)
                plsc.store_scatter(tbuf, [ivec], vals)
        l = k // P_PER_W
        p = p_base + lax.rem(k, P_PER_W)
        handles = []
        for g in range(4):
            handles.append(pltpu.async_copy(
                tbuf.at[pl.ds(1024 * g, 1024)], out_hbm.at[l, g, p], osem))
        return handles

    @pl.loop(0, N_CHUNKS, step=2)
    def _chunk(k):
        g0 = fire_gather(k, rows0, gsem0)
        g1 = fire_gather(k + 1, rows1, gsem1)
        g0.wait()
        o0 = transpose_and_out(k, rows0, tbuf0, osem0)
        g1.wait()
        o1 = transpose_and_out(k + 1, rows1, tbuf1, osem1)
        for h in o0:
            h.wait()
        for h in o1:
            h.wait()


def kernel(word, table):
    flat_word = word.reshape(N_TOTAL)
    out5 = _embed_gather(flat_word, table).reshape(L, 4, 128, 8, 128)
    return out5.transpose(2, 4, 0, 1, 3).reshape(B, L, EMBED)


# speedup vs baseline: 1.8729x; 1.1317x over previous
import functools

import jax
import jax.numpy as jnp
from jax import lax
from jax.experimental import pallas as pl
from jax.experimental.pallas import tpu as pltpu
from jax.experimental.pallas import tpu_sc as plsc

VOCAB = 1000000
EMBED = 32
B = 16384
L = 50

N_TOTAL = B * L
NW = 32
B_PER_W = B // NW
N_PER_W = B_PER_W * L
SLICE = 128
P_PER_W = B_PER_W // SLICE
N_CHUNKS = L * P_PER_W

_mesh = plsc.VectorSubcoreMesh(core_axis_name="c", subcore_axis_name="s")


@functools.partial(
    pl.kernel,
    out_type=jax.ShapeDtypeStruct((L, 4, 128, 1024), jnp.float32),
    mesh=_mesh,
    scratch_types=[
        pltpu.VMEM((N_PER_W,), jnp.int32),
        pltpu.VMEM((N_CHUNKS, SLICE), jnp.int32),
        pltpu.VMEM((SLICE, EMBED), jnp.float32),
        pltpu.VMEM((SLICE, EMBED), jnp.float32),
        pltpu.VMEM((SLICE * 33,), jnp.float32),
        pltpu.VMEM((SLICE * 33,), jnp.float32),
        pltpu.VMEM((EMBED * SLICE,), jnp.float32),
        pltpu.VMEM((EMBED * SLICE,), jnp.float32),
        pltpu.SemaphoreType.DMA,
        pltpu.SemaphoreType.DMA,
        pltpu.SemaphoreType.DMA,
        pltpu.SemaphoreType.DMA,
    ],
    compiler_params=pltpu.CompilerParams(
        use_tc_tiling_on_sc=False, needs_layout_passes=False),
)
def _embed_gather(word_hbm, table_hbm, out_hbm, idx_v, idx2_v,
                  rows0, rows1, pbuf0, pbuf1, tbuf0, tbuf1,
                  gsem0, gsem1, osem0, osem1):
    wid = lax.axis_index("s") * 2 + lax.axis_index("c")
    iota16 = lax.iota(jnp.int32, 16)
    pltpu.sync_copy(word_hbm.at[pl.ds(wid * N_PER_W, N_PER_W)], idx_v)

    @pl.loop(0, L)
    def _regroup(l):
        for p in range(P_PER_W):
            for j in range(8):
                ivec = (iota16 + (SLICE * p + 16 * j)) * L + l
                vals = plsc.load_gather(idx_v, [ivec])
                idx2_v[l * P_PER_W + p, pl.ds(16 * j, 16)] = vals

    p_base = wid * P_PER_W

    def fire_gather(k, buf, sem):
        return pltpu.async_copy(table_hbm.at[idx2_v.at[k]], buf, sem)

    piota = iota16 * 33

    def transpose_and_out(k, buf, pbuf, tbuf, osem):
        for c in range(SLICE):
            pbuf[pl.ds(33 * c, 16)] = buf[c, pl.ds(0, 16)]
            pbuf[pl.ds(33 * c + 16, 16)] = buf[c, pl.ds(16, 16)]
        for e in range(EMBED):
            for j in range(8):
                vals = plsc.load_gather(pbuf, [piota + (528 * j + e)])
                tbuf[pl.ds(SLICE * e + 16 * j, 16)] = vals
        l = k // P_PER_W
        p = p_base + lax.rem(k, P_PER_W)
        handles = []
        for g in range(4):
            handles.append(pltpu.async_copy(
                tbuf.at[pl.ds(1024 * g, 1024)], out_hbm.at[l, g, p], osem))
        return handles

    @pl.loop(0, N_CHUNKS, step=2)
    def _chunk(k):
        g0 = fire_gather(k, rows0, gsem0)
        g1 = fire_gather(k + 1, rows1, gsem1)
        g0.wait()
        o0 = transpose_and_out(k, rows0, pbuf0, tbuf0, osem0)
        g1.wait()
        o1 = transpose_and_out(k + 1, rows1, pbuf1, tbuf1, osem1)
        for h in o0:
            h.wait()
        for h in o1:
            h.wait()


def kernel(word, table):
    flat_word = word.reshape(N_TOTAL)
    out5 = _embed_gather(flat_word, table).reshape(L, 4, 128, 8, 128)
    return out5.transpose(2, 4, 0, 1, 3).reshape(B, L, EMBED)


# speedup vs baseline: 1.9938x; 1.0646x over previous
import functools

import jax
import jax.numpy as jnp
from jax import lax
from jax.experimental import pallas as pl
from jax.experimental.pallas import tpu as pltpu
from jax.experimental.pallas import tpu_sc as plsc

VOCAB = 1000000
EMBED = 32
B = 16384
L = 50

N_TOTAL = B * L
NW = 32
B_PER_W = B // NW
N_PER_W = B_PER_W * L
SLICE = 128
P_PER_W = B_PER_W // SLICE
N_CHUNKS = L * P_PER_W

_mesh = plsc.VectorSubcoreMesh(core_axis_name="c", subcore_axis_name="s")


@functools.partial(
    pl.kernel,
    out_type=jax.ShapeDtypeStruct((L, 4, 128, 1024), jnp.float32),
    mesh=_mesh,
    scratch_types=[
        pltpu.VMEM((N_PER_W,), jnp.int32),
        pltpu.VMEM((N_CHUNKS, SLICE), jnp.int32),
        pltpu.VMEM((SLICE, EMBED), jnp.float32),
        pltpu.VMEM((SLICE, EMBED), jnp.float32),
        pltpu.VMEM((SLICE * 33,), jnp.float32),
        pltpu.VMEM((SLICE * 33,), jnp.float32),
        pltpu.VMEM((EMBED * SLICE,), jnp.float32),
        pltpu.VMEM((EMBED * SLICE,), jnp.float32),
        pltpu.SemaphoreType.DMA,
        pltpu.SemaphoreType.DMA,
        pltpu.SemaphoreType.DMA,
        pltpu.SemaphoreType.DMA,
    ],
    compiler_params=pltpu.CompilerParams(
        use_tc_tiling_on_sc=False, needs_layout_passes=False),
)
def _embed_gather(word_hbm, table_hbm, out_hbm, idx_v, idx2_v,
                  rows0, rows1, pbuf0, pbuf1, tbuf0, tbuf1,
                  gsem0, gsem1, osem0, osem1):
    wid = lax.axis_index("s") * 2 + lax.axis_index("c")
    iota16 = lax.iota(jnp.int32, 16)
    pltpu.sync_copy(word_hbm.at[pl.ds(wid * N_PER_W, N_PER_W)], idx_v)

    @pl.loop(0, L)
    def _regroup(l):
        for p in range(P_PER_W):
            for j in range(8):
                ivec = (iota16 + (SLICE * p + 16 * j)) * L + l
                vals = plsc.load_gather(idx_v, [ivec])
                idx2_v[l * P_PER_W + p, pl.ds(16 * j, 16)] = vals

    p_base = wid * P_PER_W

    def fire_gather(k, buf, sem):
        return pltpu.async_copy(table_hbm.at[idx2_v.at[k]], buf, sem)

    piota = iota16 * 33

    def transpose_and_out(k, buf, pbuf, tbuf, osem):
        for c in range(SLICE):
            pbuf[pl.ds(33 * c, 16)] = buf[c, pl.ds(0, 16)]
            pbuf[pl.ds(33 * c + 16, 16)] = buf[c, pl.ds(16, 16)]
        for e in range(EMBED):
            for j in range(8):
                vals = plsc.load_gather(pbuf, [piota + (528 * j + e)])
                tbuf[pl.ds(SLICE * e + 16 * j, 16)] = vals
        l = k // P_PER_W
        p = p_base + lax.rem(k, P_PER_W)
        handles = []
        for g in range(4):
            handles.append(pltpu.async_copy(
                tbuf.at[pl.ds(1024 * g, 1024)], out_hbm.at[l, g, p], osem))
        return handles

    def drain_gather(buf, sem):
        pltpu.make_async_copy(table_hbm.at[idx2_v.at[0]], buf, sem).wait()

    fire_gather(0, rows0, gsem0)
    fire_gather(1, rows1, gsem1)

    @pl.loop(0, N_CHUNKS, step=2)
    def _chunk(k):
        drain_gather(rows0, gsem0)
        o0 = transpose_and_out(k, rows0, pbuf0, tbuf0, osem0)

        @pl.when(k + 2 < N_CHUNKS)
        def _():
            fire_gather(k + 2, rows0, gsem0)
        drain_gather(rows1, gsem1)
        o1 = transpose_and_out(k + 1, rows1, pbuf1, tbuf1, osem1)

        @pl.when(k + 3 < N_CHUNKS)
        def _():
            fire_gather(k + 3, rows1, gsem1)
        for h in o0:
            h.wait()
        for h in o1:
            h.wait()


def kernel(word, table):
    flat_word = word.reshape(N_TOTAL)
    out5 = _embed_gather(flat_word, table).reshape(L, 4, 128, 8, 128)
    return out5.transpose(2, 4, 0, 1, 3).reshape(B, L, EMBED)


# speedup vs baseline: 2.4845x; 1.2461x over previous
import functools

import jax
import jax.numpy as jnp
from jax import lax
from jax.experimental import pallas as pl
from jax.experimental.pallas import tpu as pltpu
from jax.experimental.pallas import tpu_sc as plsc

VOCAB = 1000000
EMBED = 32
B = 16384
L = 50

N_TOTAL = B * L
NW = 32
B_PER_W = B // NW
N_PER_W = B_PER_W * L
SLICE = 128
P_PER_W = B_PER_W // SLICE
N_CHUNKS = L * P_PER_W

_mesh = plsc.VectorSubcoreMesh(core_axis_name="c", subcore_axis_name="s")


@functools.partial(
    pl.kernel,
    out_type=jax.ShapeDtypeStruct((L, 4, 128, 8, 128), jnp.float32),
    mesh=_mesh,
    scratch_types=[
        pltpu.VMEM((N_PER_W,), jnp.int32),
        pltpu.VMEM((N_CHUNKS, SLICE), jnp.int32),
        pltpu.VMEM((SLICE, EMBED), jnp.float32),
        pltpu.VMEM((SLICE, EMBED), jnp.float32),
        pltpu.VMEM((EMBED, 129), jnp.float32),
        pltpu.VMEM((EMBED, 129), jnp.float32),
        pltpu.SemaphoreType.DMA,
        pltpu.SemaphoreType.DMA,
        pltpu.SemaphoreType.DMA,
        pltpu.SemaphoreType.DMA,
    ],
    compiler_params=pltpu.CompilerParams(
        use_tc_tiling_on_sc=False, needs_layout_passes=False),
)
def _embed_gather(word_hbm, table_hbm, out_hbm, idx_v, idx2_v,
                  rows0, rows1, tbuf0, tbuf1, gsem0, gsem1, osem0, osem1):
    wid = lax.axis_index("s") * 2 + lax.axis_index("c")
    iota16 = lax.iota(jnp.int32, 16)
    pltpu.sync_copy(word_hbm.at[pl.ds(wid * N_PER_W, N_PER_W)], idx_v)

    @pl.loop(0, L)
    def _regroup(l):
        for p in range(P_PER_W):
            for j in range(8):
                ivec = (iota16 + (SLICE * p + 16 * j)) * L + l
                vals = plsc.load_gather(idx_v, [ivec])
                idx2_v[l * P_PER_W + p, pl.ds(16 * j, 16)] = vals

    p_base = wid * P_PER_W

    def fire_gather(k, buf, sem):
        return pltpu.async_copy(table_hbm.at[idx2_v.at[k]], buf, sem)


    def transpose_and_out(k, buf, tbuf, osem):
        for c in range(SLICE):
            for h in range(2):
                vals = buf[c, pl.ds(16 * h, 16)]
                plsc.store_scatter(tbuf, [iota16 + 16 * h,
                                          jnp.full((16,), c, jnp.int32)], vals)
        l = k // P_PER_W
        p = p_base + lax.rem(k, P_PER_W)
        handles = []
        for g in range(4):
            handles.append(pltpu.async_copy(
                tbuf.at[pl.ds(8 * g, 8), pl.ds(0, SLICE)],
                out_hbm.at[l, g, p], osem))
        return handles

    def drain_gather(buf, sem):
        pltpu.make_async_copy(table_hbm.at[idx2_v.at[0]], buf, sem).wait()

    fire_gather(0, rows0, gsem0)
    fire_gather(1, rows1, gsem1)

    @pl.loop(0, N_CHUNKS, step=2)
    def _chunk(k):
        drain_gather(rows0, gsem0)
        o0 = transpose_and_out(k, rows0, tbuf0, osem0)

        @pl.when(k + 2 < N_CHUNKS)
        def _():
            fire_gather(k + 2, rows0, gsem0)
        drain_gather(rows1, gsem1)
        o1 = transpose_and_out(k + 1, rows1, tbuf1, osem1)

        @pl.when(k + 3 < N_CHUNKS)
        def _():
            fire_gather(k + 3, rows1, gsem1)
        for h in o0:
            h.wait()
        for h in o1:
            h.wait()


def kernel(word, table):
    flat_word = word.reshape(N_TOTAL)
    out5 = _embed_gather(flat_word, table).reshape(L, 4, 128, 8, 128)
    return out5.transpose(2, 4, 0, 1, 3).reshape(B, L, EMBED)


# speedup vs baseline: 2.4944x; 1.0040x over previous
import functools

import jax
import jax.numpy as jnp
from jax import lax
from jax.experimental import pallas as pl
from jax.experimental.pallas import tpu as pltpu
from jax.experimental.pallas import tpu_sc as plsc

VOCAB = 1000000
EMBED = 32
B = 16384
L = 50

N_TOTAL = B * L
NW = 32
B_PER_W = B // NW
N_PER_W = B_PER_W * L
SLICE = 128
P_PER_W = B_PER_W // SLICE
N_CHUNKS = L * P_PER_W

_mesh = plsc.VectorSubcoreMesh(core_axis_name="c", subcore_axis_name="s")


@functools.partial(
    pl.kernel,
    out_type=jax.ShapeDtypeStruct((L, 4, 128, 8, 128), jnp.float32),
    mesh=_mesh,
    scratch_types=[
        pltpu.VMEM((N_PER_W,), jnp.int32),
        pltpu.VMEM((N_CHUNKS, SLICE), jnp.int32),
        pltpu.VMEM((SLICE, EMBED), jnp.float32),
        pltpu.VMEM((SLICE, EMBED), jnp.float32),
        pltpu.VMEM((EMBED, 129), jnp.float32),
        pltpu.VMEM((EMBED, 129), jnp.float32),
        pltpu.SemaphoreType.DMA,
        pltpu.SemaphoreType.DMA,
        pltpu.SemaphoreType.DMA,
        pltpu.SemaphoreType.DMA,
    ],
    compiler_params=pltpu.CompilerParams(
        use_tc_tiling_on_sc=False, needs_layout_passes=False),
)
def _embed_gather(word_hbm, table_hbm, out_hbm, idx_v, idx2_v,
                  rows0, rows1, tbuf0, tbuf1, gsem0, gsem1, osem0, osem1):
    wid = lax.axis_index("s") * 2 + lax.axis_index("c")
    iota16 = lax.iota(jnp.int32, 16)
    pltpu.sync_copy(word_hbm.at[pl.ds(wid * N_PER_W, N_PER_W)], idx_v)

    @pl.loop(0, L)
    def _regroup(l):
        for p in range(P_PER_W):
            for j in range(8):
                ivec = (iota16 + (SLICE * p + 16 * j)) * L + l
                vals = plsc.load_gather(idx_v, [ivec])
                idx2_v[l * P_PER_W + p, pl.ds(16 * j, 16)] = vals

    p_base = wid * P_PER_W

    def fire_gather(k, buf, sem):
        return pltpu.async_copy(table_hbm.at[idx2_v.at[k]], buf, sem)


    def transpose_and_out(k, buf, tbuf, osem):
        for c in range(SLICE):
            for h in range(2):
                vals = buf[c, pl.ds(16 * h, 16)]
                plsc.store_scatter(tbuf, [iota16 + 16 * h,
                                          jnp.full((16,), c, jnp.int32)], vals)
        l = k // P_PER_W
        p = p_base + lax.rem(k, P_PER_W)
        handles = []
        for g in range(4):
            handles.append(pltpu.async_copy(
                tbuf.at[pl.ds(8 * g, 8), pl.ds(0, SLICE)],
                out_hbm.at[l, g, p], osem))
        return handles

    def drain_gather(buf, sem):
        pltpu.make_async_copy(table_hbm.at[idx2_v.at[0]], buf, sem).wait()

    fire_gather(0, rows0, gsem0)
    fire_gather(1, rows1, gsem1)

    @pl.loop(0, N_CHUNKS, step=2)
    def _chunk(k):
        drain_gather(rows0, gsem0)
        o0 = transpose_and_out(k, rows0, tbuf0, osem0)

        @pl.when(k + 2 < N_CHUNKS)
        def _():
            fire_gather(k + 2, rows0, gsem0)
        drain_gather(rows1, gsem1)
        o1 = transpose_and_out(k + 1, rows1, tbuf1, osem1)

        @pl.when(k + 3 < N_CHUNKS)
        def _():
            fire_gather(k + 3, rows1, gsem1)
        for h in o0:
            h.wait()
        for h in o1:
            h.wait()


TCHUNK = 2048
TSTEPS = -(-VOCAB // TCHUNK)


def _table_rm_body(x_ref, o_ref):
    t = jnp.transpose(x_ref[...], (1, 0))
    t4 = t.reshape(TCHUNK // 32, 8, 4, 32)
    o_ref[...] = jnp.concatenate(
        [t4[:, :, f, :] for f in range(4)], axis=2)


_table_rm = pl.pallas_call(
    _table_rm_body,
    grid=(TSTEPS,),
    in_specs=[pl.BlockSpec((EMBED, TCHUNK), lambda s: (0, s))],
    out_specs=pl.BlockSpec((TCHUNK // 32, 8, 128), lambda s: (s, 0, 0)),
    out_shape=jax.ShapeDtypeStruct((VOCAB // 32, 8, 128), jnp.float32),
)


def kernel(word, table):
    flat_word = word.reshape(N_TOTAL)
    table = _table_rm(table.T).reshape(VOCAB, EMBED)
    out5 = _embed_gather(flat_word, table).reshape(L, 4, 128, 8, 128)
    return out5.transpose(2, 4, 0, 1, 3).reshape(B, L, EMBED)


# speedup vs baseline: 2.6468x; 1.0611x over previous
import functools

import jax
import jax.numpy as jnp
from jax import lax
from jax.experimental import pallas as pl
from jax.experimental.pallas import tpu as pltpu
from jax.experimental.pallas import tpu_sc as plsc

VOCAB = 1000000
EMBED = 32
B = 16384
L = 50

N_TOTAL = B * L
NW = 32
B_PER_W = B // NW
N_PER_W = B_PER_W * L
SLICE = 128
P_PER_W = B_PER_W // SLICE
N_CHUNKS = L * P_PER_W

_mesh = plsc.VectorSubcoreMesh(core_axis_name="c", subcore_axis_name="s")


@functools.partial(
    pl.kernel,
    out_type=jax.ShapeDtypeStruct((L, 4, 128, 8, 128), jnp.float32),
    mesh=_mesh,
    scratch_types=[
        pltpu.VMEM((N_PER_W,), jnp.int32),
        pltpu.VMEM((N_CHUNKS, SLICE), jnp.int32),
        pltpu.VMEM((SLICE, EMBED), jnp.float32),
        pltpu.VMEM((SLICE, EMBED), jnp.float32),
        pltpu.VMEM((EMBED, 129), jnp.float32),
        pltpu.VMEM((EMBED, 129), jnp.float32),
        pltpu.SemaphoreType.DMA,
        pltpu.SemaphoreType.DMA,
        pltpu.SemaphoreType.DMA,
        pltpu.SemaphoreType.DMA,
    ],
    compiler_params=pltpu.CompilerParams(
        use_tc_tiling_on_sc=False, needs_layout_passes=False),
)
def _embed_gather(word_hbm, table_hbm, out_hbm, idx_v, idx2_v,
                  rows0, rows1, tbuf0, tbuf1, gsem0, gsem1, osem0, osem1):
    wid = lax.axis_index("s") * 2 + lax.axis_index("c")
    iota16 = lax.iota(jnp.int32, 16)
    pltpu.sync_copy(word_hbm.at[pl.ds(wid * N_PER_W, N_PER_W)], idx_v)

    @pl.loop(0, L)
    def _regroup(l):
        for p in range(P_PER_W):
            for j in range(8):
                ivec = (iota16 + (SLICE * p + 16 * j)) * L + l
                vals = plsc.load_gather(idx_v, [ivec])
                idx2_v[l * P_PER_W + p, pl.ds(16 * j, 16)] = vals

    p_base = wid * P_PER_W

    def fire_gather(k, buf, sem):
        return pltpu.async_copy(table_hbm.at[idx2_v.at[k]], buf, sem)


    def transpose_and_out(k, buf, tbuf, osem):
        for c in range(SLICE):
            for h in range(2):
                vals = buf[c, pl.ds(16 * h, 16)]
                plsc.store_scatter(tbuf, [iota16 + 16 * h,
                                          jnp.full((16,), c, jnp.int32)], vals)
        l = k // P_PER_W
        p = p_base + lax.rem(k, P_PER_W)
        handles = []
        for g in range(4):
            handles.append(pltpu.async_copy(
                tbuf.at[pl.ds(8 * g, 8), pl.ds(0, SLICE)],
                out_hbm.at[l, g, p], osem))
        return handles

    def drain_gather(buf, sem):
        pltpu.make_async_copy(table_hbm.at[idx2_v.at[0]], buf, sem).wait()

    fire_gather(0, rows0, gsem0)
    fire_gather(1, rows1, gsem1)

    @pl.loop(0, N_CHUNKS, step=2)
    def _chunk(k):
        drain_gather(rows0, gsem0)
        o0 = transpose_and_out(k, rows0, tbuf0, osem0)

        @pl.when(k + 2 < N_CHUNKS)
        def _():
            fire_gather(k + 2, rows0, gsem0)
        drain_gather(rows1, gsem1)
        o1 = transpose_and_out(k + 1, rows1, tbuf1, osem1)

        @pl.when(k + 3 < N_CHUNKS)
        def _():
            fire_gather(k + 3, rows1, gsem1)
        for h in o0:
            h.wait()
        for h in o1:
            h.wait()


TCHUNK = 4096
TSTEPS = -(-VOCAB // TCHUNK)


def _table_rm_body(x_ref, o_ref):
    eye = jnp.eye(EMBED, dtype=jnp.float32)
    t = jax.lax.dot_general(x_ref[...], eye, (((0,), (0,)), ((), ())),
                            preferred_element_type=jnp.float32)
    t4 = t.reshape(TCHUNK // 32, 8, 4, 32)
    o_ref[...] = jnp.concatenate(
        [t4[:, :, f, :] for f in range(4)], axis=2)


_table_rm = pl.pallas_call(
    _table_rm_body,
    grid=(TSTEPS,),
    in_specs=[pl.BlockSpec((EMBED, TCHUNK), lambda s: (0, s))],
    out_specs=pl.BlockSpec((TCHUNK // 32, 8, 128), lambda s: (s, 0, 0)),
    out_shape=jax.ShapeDtypeStruct((VOCAB // 32, 8, 128), jnp.float32),
)


def kernel(word, table):
    flat_word = word.reshape(N_TOTAL)
    table = _table_rm(table.T).reshape(VOCAB, EMBED)
    out5 = _embed_gather(flat_word, table).reshape(L, 4, 128, 8, 128)
    return out5.transpose(2, 4, 0, 1, 3).reshape(B, L, EMBED)
